# baseline (speedup 1.0000x reference)
import functools

import jax
import jax.numpy as jnp
from jax import lax
from jax.experimental import pallas as pl
from jax.experimental.pallas import tpu as pltpu
from jax.experimental.pallas import tpu_sc as plsc

_K = 8
_DIM = 128
_ROWS = 512
_NEG_BIG = -3e38
_PADW = 16
_NUM_WORKERS = 32


def _select_body(x3_ref, xt_ref, o_ref, t_ref, *, n_total, rows):
    b = pl.program_id(0)
    x3 = x3_ref[0]
    xt = xt_ref[0]

    sq_all = jnp.sum(x3 * x3, axis=0, keepdims=True)
    sq_t = jnp.sum(xt * xt, axis=1, keepdims=True)
    cross = lax.dot_general(xt, x3, (((1,), (0,)), ((), ())),
                            preferred_element_type=jnp.float32)
    dist = (sq_t + sq_all) - 2.0 * cross

    cols = lax.broadcasted_iota(jnp.int32, (rows, n_total), 1)

    picked = []
    for j in range(_K + 1):
        idx = jnp.argmin(dist, axis=1).astype(jnp.int32)[:, None]
        if j > 0:
            picked.append(idx)
        if j < _K:
            onehot = cols == idx
            dist = jnp.where(onehot, -_NEG_BIG, dist)
    o_ref[0] = jnp.concatenate(picked, axis=1) + b * n_total
    t_ref[0] = jnp.concatenate(
        [xt, jnp.zeros((rows, _PADW - 3), jnp.float32)], axis=1)


def _mlp_body(g_ref, xt_ref, w0_ref, w1_ref, w2_ref, o_ref, *, rows):
    nbr = g_ref[:, 0:3]
    xt = xt_ref[0]
    ctr = jnp.broadcast_to(xt[:, None, :], (rows, _K, 3)).reshape(rows * _K, 3)
    dj = nbr - ctr
    h = lax.dot_general(dj, w0_ref[...], (((1,), (0,)), ((), ())),
                        preferred_element_type=jnp.float32)
    h = jnp.maximum(h, 0.0)
    h = lax.dot_general(h, w1_ref[...], (((1,), (0,)), ((), ())),
                        preferred_element_type=jnp.float32)
    h = jnp.maximum(h, 0.0)
    h = lax.dot_general(h, w2_ref[...], (((1,), (0,)), ((), ())),
                        preferred_element_type=jnp.float32)
    h = jnp.maximum(h, 0.0)
    h3 = h.reshape(rows, _K, _DIM)
    acc = h3[:, 0, :]
    for j in range(1, _K):
        acc = jnp.maximum(acc, h3[:, j, :])
    o_ref[0] = acc.T


def kernel(x, W0, W1, W2):
    b, three, n = x.shape
    assert three == 3
    rows = _ROWS
    nt = n // rows
    items = n * _K
    per_w = items // _NUM_WORKERS
    x_t = jnp.transpose(x, (0, 2, 1))

    mesh = plsc.VectorSubcoreMesh(core_axis_name="c", subcore_axis_name="s")

    @functools.partial(
        pl.kernel, mesh=mesh,
        compiler_params=pltpu.CompilerParams(use_tc_tiling_on_sc=False),
        out_type=jax.ShapeDtypeStruct((items, _PADW), jnp.float32),
        scratch_types=[
            pltpu.VMEM((per_w,), jnp.int32),
            pltpu.VMEM((per_w, _PADW), jnp.float32),
            pltpu.SemaphoreType.DMA,
        ],
    )
    def _sc_gather(table_hbm, idx_hbm, out_hbm, idx_v, rows_v, sem):
        wid = lax.axis_index("s") * 2 + lax.axis_index("c")
        base = wid * per_w
        pltpu.sync_copy(idx_hbm.at[pl.ds(base, per_w)], idx_v)
        pltpu.async_copy(table_hbm.at[idx_v], rows_v, sem).wait()
        pltpu.sync_copy(rows_v, out_hbm.at[pl.ds(base, per_w)])

    outs = []
    for bb in range(b):
        idx, table = pl.pallas_call(
            functools.partial(_select_body, n_total=n, rows=rows),
            grid=(1, nt),
            in_specs=[
                pl.BlockSpec((1, 3, n), lambda cc, tt: (0, 0, 0)),
                pl.BlockSpec((1, rows, 3), lambda cc, tt: (0, tt, 0)),
            ],
            out_specs=[
                pl.BlockSpec((1, rows, _K), lambda cc, tt: (0, tt, 0)),
                pl.BlockSpec((1, rows, _PADW), lambda cc, tt: (0, tt, 0)),
            ],
            out_shape=[
                jax.ShapeDtypeStruct((1, n, _K), jnp.int32),
                jax.ShapeDtypeStruct((1, n, _PADW), jnp.float32),
            ],
        )(x[bb:bb + 1], x_t[bb:bb + 1])

        gathered = _sc_gather(table.reshape(n, _PADW), idx.reshape(items))

        outs.append(pl.pallas_call(
            functools.partial(_mlp_body, rows=rows),
            grid=(1, nt),
            in_specs=[
                pl.BlockSpec((rows * _K, _PADW), lambda cc, tt: (tt, 0)),
                pl.BlockSpec((1, rows, 3), lambda cc, tt: (0, tt, 0)),
                pl.BlockSpec((3, _DIM), lambda cc, tt: (0, 0)),
                pl.BlockSpec((_DIM, _DIM), lambda cc, tt: (0, 0)),
                pl.BlockSpec((_DIM, _DIM), lambda cc, tt: (0, 0)),
            ],
            out_specs=pl.BlockSpec((1, _DIM, rows), lambda cc, tt: (0, 0, tt)),
            out_shape=jax.ShapeDtypeStruct((1, _DIM, n), jnp.float32),
        )(gathered, x_t[bb:bb + 1], W0.T, W1.T, W2.T))
    return jnp.concatenate(outs, axis=0)


# speedup vs baseline: 1.0833x; 1.0833x over previous
import functools

import jax
import jax.numpy as jnp
from jax import lax
from jax.experimental import pallas as pl
from jax.experimental.pallas import tpu as pltpu
from jax.experimental.pallas import tpu_sc as plsc

_K = 8
_DIM = 128
_ROWS = 1024
_NEG_BIG = -3e38
_PADW = 16
_NUM_WORKERS = 32


def _select_body(x3_ref, xt_ref, o_ref, t_ref, *, n_total, rows):
    b = pl.program_id(0)
    x3 = x3_ref[0]
    xt = xt_ref[0]

    sq_all = jnp.sum(x3 * x3, axis=0, keepdims=True)
    sq_t = jnp.sum(xt * xt, axis=1, keepdims=True)
    cross = lax.dot_general(xt, x3, (((1,), (0,)), ((), ())),
                            preferred_element_type=jnp.float32)
    dist = (sq_t + sq_all) - 2.0 * cross

    cols = lax.broadcasted_iota(jnp.int32, (rows, n_total), 1)

    picked = []
    for j in range(_K + 1):
        idx = jnp.argmin(dist, axis=1).astype(jnp.int32)[:, None]
        if j > 0:
            picked.append(idx)
        if j < _K:
            onehot = cols == idx
            dist = jnp.where(onehot, -_NEG_BIG, dist)
    o_ref[0] = jnp.concatenate(picked, axis=1) + b * n_total
    t_ref[0] = jnp.concatenate(
        [xt, jnp.zeros((rows, _PADW - 3), jnp.float32)], axis=1)


def _mlp_body(g_ref, xt_ref, w0_ref, w1_ref, w2_ref, o_ref, *, rows):
    nbr = g_ref[:, 0:3]
    xt = xt_ref[0]
    ctr = jnp.broadcast_to(xt[:, None, :], (rows, _K, 3)).reshape(rows * _K, 3)
    dj = nbr - ctr
    h = lax.dot_general(dj, w0_ref[...], (((1,), (0,)), ((), ())),
                        preferred_element_type=jnp.float32)
    h = jnp.maximum(h, 0.0)
    h = lax.dot_general(h, w1_ref[...], (((1,), (0,)), ((), ())),
                        preferred_element_type=jnp.float32)
    h = jnp.maximum(h, 0.0)
    h = lax.dot_general(h, w2_ref[...], (((1,), (0,)), ((), ())),
                        preferred_element_type=jnp.float32)
    h = jnp.maximum(h, 0.0)
    h3 = h.reshape(rows, _K, _DIM)
    acc = h3[:, 0, :]
    for j in range(1, _K):
        acc = jnp.maximum(acc, h3[:, j, :])
    o_ref[0] = acc.T


def kernel(x, W0, W1, W2):
    b, three, n = x.shape
    assert three == 3
    rows = _ROWS
    nt = n // rows
    items = b * n * _K
    per_w = items // _NUM_WORKERS
    x_t = jnp.transpose(x, (0, 2, 1))

    idx, table = pl.pallas_call(
        functools.partial(_select_body, n_total=n, rows=rows),
        grid=(b, nt),
        in_specs=[
            pl.BlockSpec((1, 3, n), lambda bb, tt: (bb, 0, 0)),
            pl.BlockSpec((1, rows, 3), lambda bb, tt: (bb, tt, 0)),
        ],
        out_specs=[
            pl.BlockSpec((1, rows, _K), lambda bb, tt: (bb, tt, 0)),
            pl.BlockSpec((1, rows, _PADW), lambda bb, tt: (bb, tt, 0)),
        ],
        out_shape=[
            jax.ShapeDtypeStruct((b, n, _K), jnp.int32),
            jax.ShapeDtypeStruct((b, n, _PADW), jnp.float32),
        ],
    )(x, x_t)

    idx_flat = idx.reshape(items)
    table_flat = table.reshape(b * n, _PADW)

    mesh = plsc.VectorSubcoreMesh(core_axis_name="c", subcore_axis_name="s")

    @functools.partial(
        pl.kernel, mesh=mesh,
        compiler_params=pltpu.CompilerParams(use_tc_tiling_on_sc=False),
        out_type=jax.ShapeDtypeStruct((items, _PADW), jnp.float32),
        scratch_types=[
            pltpu.VMEM((per_w,), jnp.int32),
            pltpu.VMEM((per_w, _PADW), jnp.float32),
            pltpu.SemaphoreType.DMA,
        ],
    )
    def _sc_gather(table_hbm, idx_hbm, out_hbm, idx_v, rows_v, sem):
        wid = lax.axis_index("s") * 2 + lax.axis_index("c")
        base = wid * per_w
        pltpu.sync_copy(idx_hbm.at[pl.ds(base, per_w)], idx_v)
        pltpu.async_copy(table_hbm.at[idx_v], rows_v, sem).wait()
        pltpu.sync_copy(rows_v, out_hbm.at[pl.ds(base, per_w)])

    gathered = _sc_gather(table_flat, idx_flat)

    out = pl.pallas_call(
        functools.partial(_mlp_body, rows=rows),
        grid=(b, nt),
        in_specs=[
            pl.BlockSpec((rows * _K, _PADW), lambda bb, tt: (bb * nt + tt, 0)),
            pl.BlockSpec((1, rows, 3), lambda bb, tt: (bb, tt, 0)),
            pl.BlockSpec((3, _DIM), lambda bb, tt: (0, 0)),
            pl.BlockSpec((_DIM, _DIM), lambda bb, tt: (0, 0)),
            pl.BlockSpec((_DIM, _DIM), lambda bb, tt: (0, 0)),
        ],
        out_specs=pl.BlockSpec((1, _DIM, rows), lambda bb, tt: (bb, 0, tt)),
        out_shape=jax.ShapeDtypeStruct((b, _DIM, n), jnp.float32),
    )(gathered, x_t, W0.T, W1.T, W2.T)
    return out
